# Initial kernel scaffold; baseline (speedup 1.0000x reference)
import jax
import jax.numpy as jnp
from jax.experimental import pallas as pl


def kernel(node_feature, edge_index, W):
    raise NotImplementedError("write your pallas kernel here")


# speedup vs baseline: 16.8364x; 16.8364x over previous
import functools

import jax
import jax.numpy as jnp
from jax import lax
from jax.experimental import pallas as pl
from jax.experimental.pallas import tpu as pltpu
from jax.experimental.pallas import tpu_sc as plsc

N = 10000
E = 320000
D = 128

NC = 2
NS = 16
NW = NC * NS
NPAD = 10240
NPT = NPAD // NS
EPW = E // NW
K = 80
NCHUNK = EPW // K

_mesh = plsc.VectorSubcoreMesh(core_axis_name="c", subcore_axis_name="s")


@functools.partial(
    pl.kernel,
    out_type=jax.ShapeDtypeStruct((2 * NPAD,), jnp.float32),
    mesh=_mesh,
    scratch_types=[
        pltpu.VMEM((K,), jnp.int32),
        pltpu.VMEM((K,), jnp.float32),
        pltpu.VMEM_SHARED((NPAD,), jnp.float32),
    ],
)
def _deg(dst_hbm, ones_hbm, zeros_hbm, deg_out, idx_v, ones_v, deg_sh):
    c = lax.axis_index("c")
    s = lax.axis_index("s")
    wid = s * NC + c

    @pl.when(c == 0)
    def _():
        pltpu.sync_copy(ones_hbm, deg_sh.at[pl.ds(s * NPT, NPT)])

    @pl.when(c == 1)
    def _():
        pltpu.sync_copy(zeros_hbm, deg_sh.at[pl.ds(s * NPT, NPT)])

    pltpu.sync_copy(ones_hbm.at[pl.ds(0, K)], ones_v)
    plsc.subcore_barrier()

    ebase = wid * EPW

    def body(i, carry):
        pltpu.sync_copy(dst_hbm.at[pl.ds(ebase + i * K, K)], idx_v)
        pltpu.sync_copy(ones_v, deg_sh.at[idx_v], add=True)
        return carry

    lax.fori_loop(0, NCHUNK, body, 0)
    plsc.subcore_barrier()

    pltpu.sync_copy(deg_sh.at[pl.ds(s * NPT, NPT)],
                    deg_out.at[pl.ds(c * NPAD + s * NPT, NPT)])


@functools.partial(
    pl.kernel,
    out_type=jax.ShapeDtypeStruct((2 * NPAD, D), jnp.float32),
    mesh=_mesh,
    scratch_types=[
        pltpu.VMEM((K,), jnp.int32),
        pltpu.VMEM((K,), jnp.int32),
        pltpu.VMEM((K, D), jnp.float32),
        pltpu.VMEM_SHARED((NPAD, D), jnp.float32),
        pltpu.SemaphoreType.DMA,
    ],
)
def _agg(hp_hbm, src_hbm, dst_hbm, zrows_hbm, acc_out,
         idx_s, idx_d, rows, acc_sh, sem):
    c = lax.axis_index("c")
    s = lax.axis_index("s")
    wid = s * NC + c

    @pl.when(c == 0)
    def _():
        pltpu.sync_copy(hp_hbm.at[pl.ds(s * NPT, NPT)],
                        acc_sh.at[pl.ds(s * NPT, NPT)])

    @pl.when(c == 1)
    def _():
        pltpu.sync_copy(zrows_hbm, acc_sh.at[pl.ds(s * NPT, NPT)])

    plsc.subcore_barrier()

    ebase = wid * EPW

    def body(i, carry):
        pltpu.sync_copy(src_hbm.at[pl.ds(ebase + i * K, K)], idx_s)
        pltpu.sync_copy(dst_hbm.at[pl.ds(ebase + i * K, K)], idx_d)
        pltpu.async_copy(hp_hbm.at[idx_s], rows, sem).wait()
        pltpu.sync_copy(rows, acc_sh.at[idx_d], add=True)
        return carry

    lax.fori_loop(0, NCHUNK, body, 0)
    plsc.subcore_barrier()

    pltpu.sync_copy(acc_sh.at[pl.ds(s * NPT, NPT)],
                    acc_out.at[pl.ds(c * NPAD + s * NPT, NPT)])


def _scale_body(x_ref, w_ref, d0_ref, d1_ref, hp_ref, dinv_ref):
    h = jnp.dot(x_ref[...], w_ref[...], preferred_element_type=jnp.float32)
    dinv = lax.rsqrt(d0_ref[...] + d1_ref[...])
    hp_ref[...] = h * dinv
    dinv_ref[...] = dinv


_RB = 1024


def _scale(x, w, d0b, d1b):
    return pl.pallas_call(
        _scale_body,
        grid=(NPAD // _RB,),
        in_specs=[
            pl.BlockSpec((_RB, D), lambda i: (i, 0)),
            pl.BlockSpec((D, D), lambda i: (0, 0)),
            pl.BlockSpec((_RB, D), lambda i: (i, 0)),
            pl.BlockSpec((_RB, D), lambda i: (i, 0)),
        ],
        out_specs=[
            pl.BlockSpec((_RB, D), lambda i: (i, 0)),
            pl.BlockSpec((_RB, D), lambda i: (i, 0)),
        ],
        out_shape=[
            jax.ShapeDtypeStruct((NPAD, D), jnp.float32),
            jax.ShapeDtypeStruct((NPAD, D), jnp.float32),
        ],
    )(x, w, d0b, d1b)


def _combine_body(a0_ref, a1_ref, dinv_ref, out_ref):
    out_ref[...] = dinv_ref[...] * (a0_ref[...] + a1_ref[...])


def _combine(a0, a1, dinvb):
    return pl.pallas_call(
        _combine_body,
        grid=(NPAD // _RB,),
        in_specs=[
            pl.BlockSpec((_RB, D), lambda i: (i, 0)),
            pl.BlockSpec((_RB, D), lambda i: (i, 0)),
            pl.BlockSpec((_RB, D), lambda i: (i, 0)),
        ],
        out_specs=pl.BlockSpec((_RB, D), lambda i: (i, 0)),
        out_shape=jax.ShapeDtypeStruct((NPAD, D), jnp.float32),
    )(a0, a1, dinvb)


def kernel(node_feature, edge_index, W):
    src = edge_index[0]
    dst = edge_index[1]

    ones_s = jnp.ones((NPT,), jnp.float32)
    zeros_s = jnp.zeros((NPT,), jnp.float32)
    zrows_s = jnp.zeros((NPT, D), jnp.float32)

    deg2 = _deg(dst, ones_s, zeros_s).reshape(2, NPAD)
    d0b = jnp.broadcast_to(deg2[0][:, None], (NPAD, D))
    d1b = jnp.broadcast_to(deg2[1][:, None], (NPAD, D))

    x_pad = jnp.pad(node_feature, ((0, NPAD - N), (0, 0)))
    hp, dinvb = _scale(x_pad, W, d0b, d1b)

    acc2 = _agg(hp, src, dst, zrows_s).reshape(2, NPAD, D)
    out = _combine(acc2[0], acc2[1], dinvb)
    return out[:N]
